# baseline (speedup 1.0000x reference)
import functools

import jax
import jax.numpy as jnp
from jax import lax
from jax.experimental import pallas as pl
from jax.experimental.pallas import tpu as pltpu
from jax.experimental.pallas import tpu_sc as plsc

T, D, S, SE = 2048, 32, 200, 240
TOPK = 3
TB = 256
NB = T // TB
XW = 128
CH = 32
NSR = 3840
CHS = 256
NBLK = NSR // CHS
CG = 25
NCG = S // CG


def _router_body(x_ref, cl_ref, sf_ref, xs_ref, slot_ref, off_ref):
    x = x_ref[...]
    cl = cl_ref[...]
    logits = lax.dot_general(x, cl, (((1,), (1,)), ((), ())),
                             preferred_element_type=jnp.float32)
    m = jnp.max(logits, axis=1, keepdims=True)
    z = jnp.sum(jnp.exp(logits - m), axis=1, keepdims=True)
    cw = 1.0 / z
    sf2 = sf_ref[0:1, 2:3]
    sf3 = sf_ref[0:1, 3:4]
    sf4 = sf_ref[0:1, 4:5]
    scale = (sf2 * cw + sf3) * sf4
    lane = lax.broadcasted_iota(jnp.int32, (T, S), 1)
    top1 = jnp.min(jnp.where(logits == m, lane, S), axis=1, keepdims=True)
    onehot = (lane == top1).astype(jnp.float32)
    r_i = lax.broadcasted_iota(jnp.int32, (TB, TB), 0)
    c_i = lax.broadcasted_iota(jnp.int32, (TB, TB), 1)
    tril = (c_i < r_i).astype(jnp.float32)
    base = jnp.zeros((1, S), jnp.float32)
    ranks = []
    for b in range(NB):
        ob = onehot[b * TB:(b + 1) * TB, :]
        rb = lax.dot_general(tril, ob, (((1,), (0,)), ((), ())),
                             preferred_element_type=jnp.float32)
        ranks.append(jnp.sum((rb + base) * ob, axis=1, keepdims=True))
        base = base + jnp.sum(ob, axis=0, keepdims=True)
    rank = jnp.concatenate(ranks, axis=0)
    pcnt = jnp.floor((base + 7.0) / 8.0) * 8.0
    pc = jnp.concatenate([pcnt, jnp.zeros((1, 256 - S), jnp.float32)], axis=1)
    incl = pc
    for sh in (1, 2, 4, 8, 16, 32, 64, 128):
        incl = incl + jnp.concatenate(
            [jnp.zeros((1, sh), jnp.float32), incl[:, :256 - sh]], axis=1)
    aoff = incl - pc
    seg = jnp.sum(onehot * aoff[:, :S], axis=1, keepdims=True)
    slot_ref[...] = (seg + rank).astype(jnp.int32)
    off_ref[...] = aoff.astype(jnp.int32)
    xs_ref[...] = jnp.concatenate(
        [x, jnp.broadcast_to(scale, (T, XW - D))], axis=1)


SC_NW = 32
SCB = T // SC_NW


def _sc_mesh():
    return plsc.VectorSubcoreMesh(core_axis_name="c", subcore_axis_name="s")


def _sc_scatter(xs, slot1d):
    @functools.partial(
        pl.kernel, mesh=_sc_mesh(),
        out_type=jax.ShapeDtypeStruct((NSR, XW), jnp.float32),
        scratch_types=[pltpu.VMEM((SCB,), jnp.int32),
                       pltpu.VMEM((SCB, XW), jnp.float32)])
    def k(xs_hbm, idx_hbm, srt_hbm, idx_v, rows_v):
        wid = lax.axis_index("s") * 2 + lax.axis_index("c")
        base = wid * SCB
        pltpu.sync_copy(idx_hbm.at[pl.ds(base, SCB)], idx_v)
        pltpu.sync_copy(xs_hbm.at[pl.ds(base, SCB)], rows_v)
        pltpu.sync_copy(rows_v, srt_hbm.at[idx_v])
    return k(xs, slot1d)


def _sc_gather(srt, slot1d):
    @functools.partial(
        pl.kernel, mesh=_sc_mesh(),
        out_type=jax.ShapeDtypeStruct((T, XW), jnp.float32),
        scratch_types=[pltpu.VMEM((SCB,), jnp.int32),
                       pltpu.VMEM((SCB, XW), jnp.float32)])
    def k(srt_hbm, idx_hbm, out_hbm, idx_v, rows_v):
        wid = lax.axis_index("s") * 2 + lax.axis_index("c")
        base = wid * SCB
        pltpu.sync_copy(idx_hbm.at[pl.ds(base, SCB)], idx_v)
        pltpu.sync_copy(srt_hbm.at[idx_v], rows_v)
        pltpu.sync_copy(rows_v, out_hbm.at[pl.ds(base, SCB)])
    return k(srt, slot1d)


def _score_combine(xs, vc, grow, sf0, sf1, sgn):
    xc = xs[:, :D]
    sc = xs[:, D:D + 1]
    if vc.ndim == 2:
        inner = lax.dot_general(xc, vc, (((1,), (1,)), ((), ())),
                                preferred_element_type=jnp.float32)
    else:
        xc3 = xc.reshape(CG, CH, D)
        inner3 = lax.dot_general(xc3, vc, (((2,), (2,)), ((0,), (0,))),
                                 preferred_element_type=jnp.float32)
        inner = inner3.reshape(CG * CH, SE)
    cur = inner * sgn
    ms, ohs = [], []
    for _k in range(TOPK):
        mk = jnp.max(cur, axis=1, keepdims=True)
        ohk = cur == mk
        ms.append(mk)
        ohs.append(ohk)
        cur = jnp.where(ohk, -1.0e30, cur)
    gk = [jax.nn.sigmoid(sf0 * (ms[k] * sgn) + sf1) for k in range(TOPK)]
    tot = gk[0] + gk[1] + gk[2] + 1e-6
    wmat = sum(jnp.where(ohs[k], gk[k] / tot, 0.0) for k in range(TOPK))
    if vc.ndim == 2:
        wmat = wmat * grow
        outc = lax.dot_general(wmat, vc, (((1,), (0,)), ((), ())),
                               preferred_element_type=jnp.float32)
    else:
        w3 = wmat.reshape(CG, CH, SE) * grow
        outc = lax.dot_general(w3, vc, (((2,), (1,)), ((0,), (0,))),
                               preferred_element_type=jnp.float32)
        outc = outc.reshape(CG * CH, D)
    return outc * sc


def _cluster_body(off_ref, srt_ref, vec_ref, gain_ref, sf_ref, out_ref):
    s = pl.program_id(0)

    @pl.when(s == 0)
    def _init():
        out_ref[...] = jnp.zeros_like(out_ref)

    vcs = vec_ref[0]
    sf0 = sf_ref[0:1, 0:1]
    sf1 = sf_ref[0:1, 1:2]
    sgn = jnp.sign(sf0)
    offs = [off_ref[0, s * CG + g] for g in range(CG + 1)]
    xs_b = jnp.concatenate(
        [srt_ref[pl.ds(offs[g], CH), :] for g in range(CG)], axis=0)
    out_b = _score_combine(xs_b, vcs, gain_ref[...], sf0, sf1, sgn)
    out3 = out_b.reshape(CG, CH, D)
    for g in range(CG):
        out_ref[pl.ds(offs[g], CH), :D] = out3[g]
        nch = (offs[g + 1] - offs[g] + (CH - 1)) // CH
        vc_g = vcs[g]
        grow_g = gain_ref[g]

        def chunk(j, _, g=g, vc_g=vc_g, grow_g=grow_g):
            start = offs[g] + j * CH
            xs = srt_ref[pl.ds(start, CH), :]
            out_ref[pl.ds(start, CH), :D] = _score_combine(
                xs, vc_g, grow_g, sf0, sf1, sgn)
            return 0

        lax.fori_loop(1, nch, chunk, 0)


def kernel(x, vec_clusters, vec, gain_factors, sigmoid_factors, block_indices):
    cl = vec_clusters[0]
    gain0 = (gain_factors[0] @ jnp.array([1.0, 0.0], jnp.float32)
             ).reshape(S, 1, SE)
    sfp = sigmoid_factors
    xs, slot, aoff = pl.pallas_call(
        _router_body,
        out_shape=[jax.ShapeDtypeStruct((T, XW), jnp.float32),
                   jax.ShapeDtypeStruct((T, 1), jnp.int32),
                   jax.ShapeDtypeStruct((1, 256), jnp.int32)],
    )(x, cl, sfp)
    slot1d = slot.reshape(T)
    xs_srt = _sc_scatter(xs, slot1d)
    out_srt = pl.pallas_call(
        _cluster_body,
        grid=(NCG,),
        in_specs=[pl.BlockSpec(memory_space=pltpu.SMEM),
                  pl.BlockSpec((NSR, XW), lambda c: (0, 0)),
                  pl.BlockSpec((1, CG, SE, D), lambda c: (0, c, 0, 0)),
                  pl.BlockSpec((CG, 1, SE), lambda c: (c, 0, 0)),
                  pl.BlockSpec((7, 5), lambda c: (0, 0))],
        out_specs=pl.BlockSpec((NSR, XW), lambda c: (0, 0)),
        out_shape=jax.ShapeDtypeStruct((NSR, XW), jnp.float32),
    )(aoff, xs_srt, vec, gain0, sfp)
    out = _sc_gather(out_srt, slot1d)
    return out[:, :D]


# speedup vs baseline: 2.2553x; 2.2553x over previous
import functools

import jax
import jax.numpy as jnp
from jax import lax
from jax.experimental import pallas as pl
from jax.experimental.pallas import tpu as pltpu
from jax.experimental.pallas import tpu_sc as plsc

T, D, S, SE = 2048, 32, 200, 240
TOPK = 3
TB = 256
NB = T // TB
XW = 128
CH = 32
NSR = 3840
CHS = 256
NBLK = NSR // CHS
CG = 25
NCG = S // CG


def _router_body(x_ref, cl_ref, sf_ref, xs_ref, slot_ref, off_ref):
    x = x_ref[...]
    cl = cl_ref[...]
    logits = lax.dot_general(x, cl, (((1,), (1,)), ((), ())),
                             preferred_element_type=jnp.float32)
    m = jnp.max(logits, axis=1, keepdims=True)
    z = jnp.sum(jnp.exp(logits - m), axis=1, keepdims=True)
    cw = 1.0 / z
    sf2 = sf_ref[0:1, 2:3]
    sf3 = sf_ref[0:1, 3:4]
    sf4 = sf_ref[0:1, 4:5]
    scale = (sf2 * cw + sf3) * sf4
    lane = lax.broadcasted_iota(jnp.int32, (T, S), 1)
    top1 = jnp.min(jnp.where(logits == m, lane, S), axis=1, keepdims=True)
    onehot = (lane == top1).astype(jnp.float32)
    r_i = lax.broadcasted_iota(jnp.int32, (TB, TB), 0)
    c_i = lax.broadcasted_iota(jnp.int32, (TB, TB), 1)
    tril = (c_i < r_i).astype(jnp.float32)
    base = jnp.zeros((1, S), jnp.float32)
    ranks = []
    for b in range(NB):
        ob = onehot[b * TB:(b + 1) * TB, :]
        rb = lax.dot_general(tril, ob, (((1,), (0,)), ((), ())),
                             preferred_element_type=jnp.float32)
        ranks.append(jnp.sum((rb + base) * ob, axis=1, keepdims=True))
        base = base + jnp.sum(ob, axis=0, keepdims=True)
    rank = jnp.concatenate(ranks, axis=0)
    pcnt = jnp.floor((base + 7.0) / 8.0) * 8.0
    pc = jnp.concatenate([pcnt, jnp.zeros((1, 256 - S), jnp.float32)], axis=1)
    incl = pc
    for sh in (1, 2, 4, 8, 16, 32, 64, 128):
        incl = incl + jnp.concatenate(
            [jnp.zeros((1, sh), jnp.float32), incl[:, :256 - sh]], axis=1)
    aoff = incl - pc
    seg = jnp.sum(onehot * aoff[:, :S], axis=1, keepdims=True)
    slot_ref[...] = (seg + rank).astype(jnp.int32)
    off_ref[...] = aoff.astype(jnp.int32)
    xs_ref[...] = jnp.concatenate(
        [x, jnp.broadcast_to(scale, (T, XW - D))], axis=1)


SC_NW = 32
SCB = T // SC_NW


def _sc_mesh():
    return plsc.VectorSubcoreMesh(core_axis_name="c", subcore_axis_name="s")


def _sc_scatter(xs, slot1d):
    @functools.partial(
        pl.kernel, mesh=_sc_mesh(),
        out_type=jax.ShapeDtypeStruct((NSR, XW), jnp.float32),
        scratch_types=[pltpu.VMEM((SCB,), jnp.int32),
                       pltpu.VMEM((SCB, XW), jnp.float32)])
    def k(xs_hbm, idx_hbm, srt_hbm, idx_v, rows_v):
        wid = lax.axis_index("s") * 2 + lax.axis_index("c")
        base = wid * SCB
        pltpu.sync_copy(idx_hbm.at[pl.ds(base, SCB)], idx_v)
        pltpu.sync_copy(xs_hbm.at[pl.ds(base, SCB)], rows_v)
        pltpu.sync_copy(rows_v, srt_hbm.at[idx_v])
    return k(xs, slot1d)


def _sc_gather(srt, slot1d):
    @functools.partial(
        pl.kernel, mesh=_sc_mesh(),
        out_type=jax.ShapeDtypeStruct((T, XW), jnp.float32),
        scratch_types=[pltpu.VMEM((SCB,), jnp.int32),
                       pltpu.VMEM((SCB, XW), jnp.float32)])
    def k(srt_hbm, idx_hbm, out_hbm, idx_v, rows_v):
        wid = lax.axis_index("s") * 2 + lax.axis_index("c")
        base = wid * SCB
        pltpu.sync_copy(idx_hbm.at[pl.ds(base, SCB)], idx_v)
        pltpu.sync_copy(srt_hbm.at[idx_v], rows_v)
        pltpu.sync_copy(rows_v, out_hbm.at[pl.ds(base, SCB)])
    return k(srt, slot1d)


def _score_combine(xs, vc, grow, sf0, sf1, sgn):
    xc = xs[:, :D]
    sc = xs[:, D:D + 1]
    if vc.ndim == 2:
        inner = lax.dot_general(xc, vc, (((1,), (1,)), ((), ())),
                                preferred_element_type=jnp.float32)
    else:
        xc3 = xc.reshape(CG, CH, D)
        inner3 = lax.dot_general(xc3, vc, (((2,), (2,)), ((0,), (0,))),
                                 preferred_element_type=jnp.float32)
        inner = inner3.reshape(CG * CH, SE)
    cur = inner * sgn
    ms, ohs = [], []
    for _k in range(TOPK):
        mk = jnp.max(cur, axis=1, keepdims=True)
        ohk = cur == mk
        ms.append(mk)
        ohs.append(ohk)
        cur = jnp.where(ohk, -1.0e30, cur)
    gk = [jax.nn.sigmoid(sf0 * (ms[k] * sgn) + sf1) for k in range(TOPK)]
    tot = gk[0] + gk[1] + gk[2] + 1e-6
    wmat = sum(jnp.where(ohs[k], gk[k] / tot, 0.0) for k in range(TOPK))
    if vc.ndim == 2:
        wmat = wmat * grow
        outc = lax.dot_general(wmat, vc, (((1,), (0,)), ((), ())),
                               preferred_element_type=jnp.float32)
    else:
        w3 = wmat.reshape(CG, CH, SE) * grow
        outc = lax.dot_general(w3, vc, (((2,), (1,)), ((0,), (0,))),
                               preferred_element_type=jnp.float32)
        outc = outc.reshape(CG * CH, D)
    return outc * sc


def _cluster_body(off_ref, srt_ref, vec_ref, gain_ref, sf_ref, out_ref):
    s = pl.program_id(0)

    @pl.when(s == 0)
    def _init():
        out_ref[...] = jnp.zeros_like(out_ref)

    vcs = vec_ref[...]
    sf0 = sf_ref[0:1, 0:1]
    sf1 = sf_ref[0:1, 1:2]
    sgn = jnp.sign(sf0)
    offs = [off_ref[0, s * CG + g] for g in range(CG + 1)]
    xs_b = jnp.concatenate(
        [srt_ref[pl.ds(offs[g], CH), :] for g in range(CG)], axis=0)
    out_b = _score_combine(xs_b, vcs, gain_ref[...], sf0, sf1, sgn)
    out3 = out_b.reshape(CG, CH, D)
    for g in range(CG):
        out_ref[pl.ds(offs[g], CH), :D] = out3[g]
        nch = (offs[g + 1] - offs[g] + (CH - 1)) // CH
        vc_g = vcs[g]
        grow_g = gain_ref[g]

        def chunk(j, _, g=g, vc_g=vc_g, grow_g=grow_g):
            start = offs[g] + j * CH
            xs = srt_ref[pl.ds(start, CH), :]
            out_ref[pl.ds(start, CH), :D] = _score_combine(
                xs, vc_g, grow_g, sf0, sf1, sgn)
            return 0

        lax.fori_loop(1, nch, chunk, 0)


def kernel(x, vec_clusters, vec, gain_factors, sigmoid_factors, block_indices):
    cl = vec_clusters[0]
    vec0 = vec[0]
    gain0 = (gain_factors[0] @ jnp.array([1.0, 0.0], jnp.float32)
             ).reshape(S, 1, SE)
    sfp = sigmoid_factors
    xs, slot, aoff = pl.pallas_call(
        _router_body,
        out_shape=[jax.ShapeDtypeStruct((T, XW), jnp.float32),
                   jax.ShapeDtypeStruct((T, 1), jnp.int32),
                   jax.ShapeDtypeStruct((1, 256), jnp.int32)],
    )(x, cl, sfp)
    slot1d = slot.reshape(T)
    xs_srt = _sc_scatter(xs, slot1d)
    out_srt = pl.pallas_call(
        _cluster_body,
        grid=(NCG,),
        in_specs=[pl.BlockSpec(memory_space=pltpu.SMEM),
                  pl.BlockSpec((NSR, XW), lambda c: (0, 0)),
                  pl.BlockSpec((CG, SE, D), lambda c: (c, 0, 0)),
                  pl.BlockSpec((CG, 1, SE), lambda c: (c, 0, 0)),
                  pl.BlockSpec((7, 5), lambda c: (0, 0))],
        out_specs=pl.BlockSpec((NSR, XW), lambda c: (0, 0)),
        out_shape=jax.ShapeDtypeStruct((NSR, XW), jnp.float32),
    )(aoff, xs_srt, vec0, gain0, sfp)
    out = _sc_gather(out_srt, slot1d)
    return out[:, :D]
